# baseline (speedup 1.0000x reference)
import functools

import jax
import jax.numpy as jnp
from jax import lax
from jax.experimental import pallas as pl
from jax.experimental.pallas import tpu as pltpu
from jax.experimental.pallas import tpu_sc as plsc

_N = 10000
_D = 128
_E = 320000
_NC = 2
_NS = 16
_NW = _NC * _NS
_NPAD = 10240
_RPS = _NPAD // _NS
_NOUT = 10240

_mesh = plsc.VectorSubcoreMesh(core_axis_name="c", subcore_axis_name="s")


_EPT = _E // _NW
_CHD = 128
_CPTD = 80
_CHA = 120
_CPTA = 84
_EPA = _CPTA * _CHA


@functools.partial(
    pl.kernel,
    mesh=_mesh,
    out_type=jax.ShapeDtypeStruct((_NC, _NOUT), jnp.float32),
    scratch_types=[
        pltpu.VMEM((_CPTD, _CHD), jnp.int32),
        pltpu.VMEM((_CHD,), jnp.float32),
        pltpu.VMEM((640,), jnp.float32),
        pltpu.VMEM_SHARED((_NPAD,), jnp.float32),
    ],
)
def _deg_kernel(dsti_hbm, out_hbm, dsti_v, ones_v, zb, acc):
    c = lax.axis_index("c")
    s = lax.axis_index("s")
    w = c * _NS + s
    for j in range(_CHD // 16):
        ones_v[pl.ds(16 * j, 16)] = jnp.ones((16,), jnp.float32)

    def zfill(i, carry):
        zb[pl.ds(16 * i, 16)] = jnp.zeros((16,), jnp.float32)
        return carry

    lax.fori_loop(0, 640 // 16, zfill, 0)
    pltpu.sync_copy(zb.at[pl.ds(0, _RPS)], acc.at[pl.ds(s * _RPS, _RPS)])
    pltpu.sync_copy(dsti_hbm.at[w], dsti_v)
    plsc.subcore_barrier()

    def body(k, carry):
        pltpu.sync_copy(ones_v, acc.at[dsti_v.at[k]], add=True)
        return carry

    lax.fori_loop(0, _CPTD, body, 0)
    plsc.subcore_barrier()
    pltpu.sync_copy(acc.at[pl.ds(s * _RPS, _RPS)],
                    out_hbm.at[c, pl.ds(s * _RPS, _RPS)])


@functools.partial(
    pl.kernel,
    mesh=_mesh,
    out_type=jax.ShapeDtypeStruct((_NC, _NOUT, _D), jnp.float32),
    scratch_types=(
        [pltpu.VMEM((_CHA,), jnp.int32)] * 3
        + [pltpu.VMEM((_CHA,), jnp.int32)] * 3
        + [pltpu.VMEM((_CHA, _D), jnp.float32)] * 3
        + [pltpu.VMEM_SHARED((_NPAD, _D), jnp.float32)]
        + [pltpu.SemaphoreType.DMA] * 12
    ),
)
def _agg_kernel(hp_hbm, srcf_hbm, dstf_hbm, out_hbm, *refs):
    srcb = refs[0:3]
    dstb = refs[3:6]
    rows = refs[6:9]
    acc = refs[9]
    si = refs[10:13]
    di = refs[13:16]
    gs = refs[16:19]
    ss = refs[19:22]
    c = lax.axis_index("c")
    s = lax.axis_index("s")
    w = c * _NS + s

    def zfill(i, carry):
        for j in range(_D // 16):
            rows[0][i, pl.ds(16 * j, 16)] = jnp.zeros((16,), jnp.float32)
        return carry

    lax.fori_loop(0, _CHA, zfill, 0)
    for r in range((_RPS + _CHA - 1) // _CHA):
        base = min(r * _CHA, _RPS - _CHA)
        pltpu.sync_copy(rows[0], acc.at[pl.ds(s * _RPS + base, _CHA)])

    ebase = w * _EPA

    def idxload(hbm, k, buf, sem):
        pltpu.async_copy(hbm.at[pl.ds(ebase + k * _CHA, _CHA)], buf, sem)

    def idx_wait(buf, sem):
        pltpu.make_async_copy(dstf_hbm.at[pl.ds(0, _CHA)], buf, sem).wait()

    def gather(b):
        pltpu.async_copy(hp_hbm.at[srcb[b]], rows[b], gs[b])

    def gather_wait(b):
        pltpu.make_async_copy(hp_hbm.at[srcb[b]], rows[b], gs[b]).wait()

    def scat_wait(b):
        pltpu.make_async_copy(rows[b], acc.at[dstb[b]], ss[b]).wait()

    for b in range(3):
        idxload(srcf_hbm, b, srcb[b], si[b])
        idxload(dstf_hbm, b, dstb[b], di[b])
    plsc.subcore_barrier()
    for b in range(3):
        idx_wait(srcb[b], si[b])
        gather(b)

    def body(i, carry):
        for b in range(3):
            kn = jnp.minimum(3 * i + 3 + b, _CPTA - 1)
            idx_wait(dstb[b], di[b])
            gather_wait(b)
            pltpu.async_copy(rows[b], acc.at[dstb[b]], ss[b], add=True)
            idxload(srcf_hbm, kn, srcb[b], si[b])
        for b in range(3):
            kn = jnp.minimum(3 * i + 3 + b, _CPTA - 1)
            scat_wait(b)
            idxload(dstf_hbm, kn, dstb[b], di[b])
            idx_wait(srcb[b], si[b])
            gather(b)
        return carry

    lax.fori_loop(0, _CPTA // 3 - 1, body, 0)
    for b in range(3):
        idx_wait(dstb[b], di[b])
        gather_wait(b)
        pltpu.sync_copy(rows[b], acc.at[dstb[b]], add=True)
    plsc.subcore_barrier()
    pltpu.sync_copy(acc.at[pl.ds(s * _RPS, _RPS)],
                    out_hbm.at[c, pl.ds(s * _RPS, _RPS)])


def _tc1a_body(x_ref, w_ref, h_ref):
    h_ref[...] = jnp.dot(x_ref[...], w_ref[...],
                         preferred_element_type=jnp.float32)


_tc1a = pl.pallas_call(
    _tc1a_body,
    grid=(10,),
    in_specs=[pl.BlockSpec((1000, _D), lambda i: (i, 0)),
              pl.BlockSpec((_D, _D), lambda i: (0, 0))],
    out_specs=pl.BlockSpec((1000, _D), lambda i: (i, 0)),
    out_shape=jax.ShapeDtypeStruct((_N, _D), jnp.float32),
)


def _dinv_block(dp):
    rows8 = lax.rsqrt(dp[0] + dp[1] + 1.0)
    r0 = lax.broadcasted_iota(jnp.int32, (_D, _D), 0)
    c0 = lax.broadcasted_iota(jnp.int32, (_D, _D), 1)
    eye = (r0 == c0)
    groups = []
    for g in range(8):
        row = rows8[g:g + 1]
        m = jnp.where(eye, jnp.broadcast_to(row, (_D, _D)), 0.0)
        col = jnp.sum(m, axis=1, keepdims=True)
        groups.append(jnp.broadcast_to(col, (_D, _D)))
    return jnp.concatenate(groups, axis=0)


def _tc1b_body(h_ref, degp_ref, hp_ref):
    hp_ref[...] = h_ref[...] * _dinv_block(degp_ref[...])


_tc1b = pl.pallas_call(
    _tc1b_body,
    grid=(10,),
    in_specs=[pl.BlockSpec((1024, _D), lambda i: (i, 0)),
              pl.BlockSpec((2, 8, _D), lambda i: (0, i, 0))],
    out_specs=pl.BlockSpec((1024, _D), lambda i: (i, 0)),
    out_shape=jax.ShapeDtypeStruct((_N, _D), jnp.float32),
)


def _tc2_body(aggp_ref, hp_ref, degp_ref, b_ref, res_ref, w_ref,
              h_ref, hpn_ref):
    dinv = _dinv_block(degp_ref[...])
    agg = aggp_ref[0] + aggp_ref[1]
    z = dinv * (agg + hp_ref[...]) + b_ref[...]
    h = jnp.maximum(z, 0.0) + res_ref[...]
    h_ref[...] = h
    hpn_ref[...] = jnp.dot(h, w_ref[...],
                           preferred_element_type=jnp.float32) * dinv


_tc2 = pl.pallas_call(
    _tc2_body,
    grid=(10,),
    in_specs=[pl.BlockSpec((2, 1024, _D), lambda i: (0, i, 0)),
              pl.BlockSpec((1024, _D), lambda i: (i, 0)),
              pl.BlockSpec((2, 8, _D), lambda i: (0, i, 0)),
              pl.BlockSpec((1, _D), lambda i: (0, 0)),
              pl.BlockSpec((1024, _D), lambda i: (i, 0)),
              pl.BlockSpec((_D, _D), lambda i: (0, 0))],
    out_specs=(pl.BlockSpec((1024, _D), lambda i: (i, 0)),
               pl.BlockSpec((1024, _D), lambda i: (i, 0))),
    out_shape=(jax.ShapeDtypeStruct((_N, _D), jnp.float32),
               jax.ShapeDtypeStruct((_N, _D), jnp.float32)),
)


def _tc3_body(aggp_ref, hp_ref, degp_ref, b_ref, res_ref, wout_ref,
              bout_ref, out_ref):
    dinv = _dinv_block(degp_ref[...])
    agg = aggp_ref[0] + aggp_ref[1]
    z = dinv * (agg + hp_ref[...]) + b_ref[...]
    h = jnp.maximum(z, 0.0) + res_ref[...]
    out_ref[...] = jnp.dot(h, wout_ref[...],
                           preferred_element_type=jnp.float32) + bout_ref[...]


_tc3 = pl.pallas_call(
    _tc3_body,
    grid=(10,),
    in_specs=[pl.BlockSpec((2, 1024, _D), lambda i: (0, i, 0)),
              pl.BlockSpec((1024, _D), lambda i: (i, 0)),
              pl.BlockSpec((2, 8, _D), lambda i: (0, i, 0)),
              pl.BlockSpec((1, _D), lambda i: (0, 0)),
              pl.BlockSpec((1024, _D), lambda i: (i, 0)),
              pl.BlockSpec((_D, _D), lambda i: (0, 0)),
              pl.BlockSpec((1, _D), lambda i: (0, 0))],
    out_specs=pl.BlockSpec((1024, _D), lambda i: (i, 0)),
    out_shape=jax.ShapeDtypeStruct((_N, _D), jnp.float32),
)


def kernel(x, edge_index, W1, b1, W2, b2, W_out, b_out):
    src2 = edge_index[0].reshape(_NW, _EPT)
    dst2 = edge_index[1].reshape(_NW, _EPT)
    npd = _CPTD * _CHD - _EPT
    dpad = jnp.full((_NW, npd), _N, jnp.int32)
    dstpad = jnp.concatenate([dst2, dpad], axis=1).reshape(_NW, _CPTD, _CHD)
    npa = _EPA - _EPT
    spada = jnp.zeros((_NW, npa), jnp.int32)
    dpada = jnp.full((_NW, npa), _N, jnp.int32)
    srcf = jnp.concatenate([src2, spada], axis=1).reshape(_NW * _EPA)
    dstf = jnp.concatenate([dst2, dpada], axis=1).reshape(_NW * _EPA)

    degp = _deg_kernel(dstpad)
    h1raw = _tc1a(x, W1)

    degp_r = degp.reshape(_NC, _NOUT // _D, _D)
    hp1 = _tc1b(h1raw, degp_r)
    agg1 = _agg_kernel(hp1, srcf, dstf)
    h1, hp2 = _tc2(agg1, hp1, degp_r, b1.reshape(1, _D), x, W2)
    agg2 = _agg_kernel(hp2, srcf, dstf)
    out = _tc3(agg2, hp2, degp_r, b2.reshape(1, _D), h1, W_out,
               b_out.reshape(1, _D))
    return out


# speedup vs baseline: 1.6140x; 1.6140x over previous
import functools

import jax
import jax.numpy as jnp
from jax import lax
from jax.experimental import pallas as pl
from jax.experimental.pallas import tpu as pltpu
from jax.experimental.pallas import tpu_sc as plsc

_N = 10000
_D = 128
_E = 320000
_NC = 2
_NS = 16
_NW = _NC * _NS
_CH = 80
_CPT = _E // _NW // _CH
_NPAD = 10240
_RPS = _NPAD // _NS

_mesh = plsc.VectorSubcoreMesh(core_axis_name="c", subcore_axis_name="s")


_EPT = _E // _NW
_CHD = 128
_CPTD = 80


@functools.partial(
    pl.kernel,
    mesh=_mesh,
    out_type=jax.ShapeDtypeStruct((_NC, _NPAD), jnp.float32),
    scratch_types=[
        pltpu.VMEM((_CPTD, _CHD), jnp.int32),
        pltpu.VMEM((_CHD,), jnp.float32),
        pltpu.VMEM((_RPS,), jnp.float32),
        pltpu.VMEM_SHARED((_NPAD,), jnp.float32),
    ],
)
def _deg_kernel(dsti_hbm, out_hbm, dsti_v, ones_v, zb, acc):
    c = lax.axis_index("c")
    s = lax.axis_index("s")
    w = c * _NS + s
    for j in range(_CHD // 16):
        ones_v[pl.ds(16 * j, 16)] = jnp.ones((16,), jnp.float32)

    def zfill(i, carry):
        zb[pl.ds(16 * i, 16)] = jnp.zeros((16,), jnp.float32)
        return carry

    lax.fori_loop(0, _RPS // 16, zfill, 0)
    pltpu.sync_copy(zb, acc.at[pl.ds(s * _RPS, _RPS)])
    pltpu.sync_copy(dsti_hbm.at[w], dsti_v)
    plsc.subcore_barrier()

    def body(k, carry):
        pltpu.sync_copy(ones_v, acc.at[dsti_v.at[k]], add=True)
        return carry

    lax.fori_loop(0, _CPTD, body, 0)
    plsc.subcore_barrier()
    pltpu.sync_copy(acc.at[pl.ds(s * _RPS, _RPS)],
                    out_hbm.at[c, pl.ds(s * _RPS, _RPS)])


@functools.partial(
    pl.kernel,
    mesh=_mesh,
    out_type=jax.ShapeDtypeStruct((_NC, _NPAD, _D), jnp.float32),
    scratch_types=(
        [pltpu.VMEM((_CH,), jnp.int32)] * 4
        + [pltpu.VMEM((_CH,), jnp.int32)] * 4
        + [pltpu.VMEM((_CH, _D), jnp.float32)] * 4
        + [pltpu.VMEM_SHARED((_NPAD, _D), jnp.float32)]
        + [pltpu.SemaphoreType.DMA] * 16
    ),
)
def _agg_kernel(hp_hbm, ei_hbm, out_hbm, *refs):
    srcb = refs[0:4]
    dstb = refs[4:8]
    rows = refs[8:12]
    acc = refs[12]
    si = refs[13:17]
    di = refs[17:21]
    gs = refs[21:25]
    ss = refs[25:29]
    c = lax.axis_index("c")
    s = lax.axis_index("s")
    w = c * _NS + s

    def zfill(i, carry):
        for j in range(_D // 16):
            rows[0][i, pl.ds(16 * j, 16)] = jnp.zeros((16,), jnp.float32)
        return carry

    lax.fori_loop(0, _CH, zfill, 0)
    for r in range(_RPS // _CH):
        pltpu.sync_copy(rows[0], acc.at[pl.ds(s * _RPS + r * _CH, _CH)])
    ebase = w * (_CPT * _CH)

    def idxload(half, k, buf, sem):
        pltpu.async_copy(
            ei_hbm.at[pl.ds(half * _E + ebase + k * _CH, _CH)], buf, sem)

    def idx_wait(half, buf, sem):
        pltpu.make_async_copy(ei_hbm.at[pl.ds(0, _CH)], buf, sem).wait()

    def gather(b):
        pltpu.async_copy(hp_hbm.at[srcb[b]], rows[b], gs[b])

    def gather_wait(b):
        pltpu.make_async_copy(hp_hbm.at[srcb[b]], rows[b], gs[b]).wait()

    def scat_wait(b):
        pltpu.make_async_copy(rows[b], acc.at[dstb[b]], ss[b]).wait()

    for b in range(4):
        idxload(0, b, srcb[b], si[b])
        idxload(1, b, dstb[b], di[b])
    plsc.subcore_barrier()
    for b in range(4):
        idx_wait(0, srcb[b], si[b])
        gather(b)

    def body(i, carry):
        for b in range(4):
            kn = jnp.minimum(4 * i + 4 + b, _CPT - 1)
            idx_wait(1, dstb[b], di[b])
            gather_wait(b)
            pltpu.async_copy(rows[b], acc.at[dstb[b]], ss[b], add=True)
            idxload(0, kn, srcb[b], si[b])
        for b in range(4):
            kn = jnp.minimum(4 * i + 4 + b, _CPT - 1)
            scat_wait(b)
            idxload(1, kn, dstb[b], di[b])
            idx_wait(0, srcb[b], si[b])
            gather(b)
        return carry

    lax.fori_loop(0, (_CPT - 1) // 4, body, 0)
    idx_wait(1, dstb[0], di[0])
    gather_wait(0)
    pltpu.sync_copy(rows[0], acc.at[dstb[0]], add=True)
    for b in range(1, 4):
        idx_wait(1, dstb[b], di[b])
        gather_wait(b)
    plsc.subcore_barrier()
    pltpu.sync_copy(acc.at[pl.ds(s * _RPS, _RPS)],
                    out_hbm.at[c, pl.ds(s * _RPS, _RPS)])


def _tc1a_body(x_ref, w_ref, h_ref):
    h_ref[...] = jnp.dot(x_ref[...], w_ref[...],
                         preferred_element_type=jnp.float32)


_tc1a = pl.pallas_call(
    _tc1a_body,
    grid=(10,),
    in_specs=[pl.BlockSpec((1000, _D), lambda i: (i, 0)),
              pl.BlockSpec((_D, _D), lambda i: (0, 0))],
    out_specs=pl.BlockSpec((1000, _D), lambda i: (i, 0)),
    out_shape=jax.ShapeDtypeStruct((_N, _D), jnp.float32),
)


def _dinv_block(dp):
    rows8 = lax.rsqrt(dp[0] + dp[1] + 1.0)
    r0 = lax.broadcasted_iota(jnp.int32, (_D, _D), 0)
    c0 = lax.broadcasted_iota(jnp.int32, (_D, _D), 1)
    eye = (r0 == c0)
    groups = []
    for g in range(8):
        row = rows8[g:g + 1]
        m = jnp.where(eye, jnp.broadcast_to(row, (_D, _D)), 0.0)
        col = jnp.sum(m, axis=1, keepdims=True)
        groups.append(jnp.broadcast_to(col, (_D, _D)))
    return jnp.concatenate(groups, axis=0)


def _tc1b_body(h_ref, degp_ref, hp_ref):
    hp_ref[...] = h_ref[...] * _dinv_block(degp_ref[...])


_tc1b = pl.pallas_call(
    _tc1b_body,
    grid=(10,),
    in_specs=[pl.BlockSpec((1024, _D), lambda i: (i, 0)),
              pl.BlockSpec((2, 8, _D), lambda i: (0, i, 0))],
    out_specs=pl.BlockSpec((1024, _D), lambda i: (i, 0)),
    out_shape=jax.ShapeDtypeStruct((_N, _D), jnp.float32),
)


def _tc2_body(aggp_ref, hp_ref, degp_ref, b_ref, res_ref, w_ref,
              h_ref, hpn_ref):
    dinv = _dinv_block(degp_ref[...])
    agg = aggp_ref[0] + aggp_ref[1]
    z = dinv * (agg + hp_ref[...]) + b_ref[...]
    h = jnp.maximum(z, 0.0) + res_ref[...]
    h_ref[...] = h
    hpn_ref[...] = jnp.dot(h, w_ref[...],
                           preferred_element_type=jnp.float32) * dinv


_tc2 = pl.pallas_call(
    _tc2_body,
    grid=(10,),
    in_specs=[pl.BlockSpec((2, 1024, _D), lambda i: (0, i, 0)),
              pl.BlockSpec((1024, _D), lambda i: (i, 0)),
              pl.BlockSpec((2, 8, _D), lambda i: (0, i, 0)),
              pl.BlockSpec((1, _D), lambda i: (0, 0)),
              pl.BlockSpec((1024, _D), lambda i: (i, 0)),
              pl.BlockSpec((_D, _D), lambda i: (0, 0))],
    out_specs=(pl.BlockSpec((1024, _D), lambda i: (i, 0)),
               pl.BlockSpec((1024, _D), lambda i: (i, 0))),
    out_shape=(jax.ShapeDtypeStruct((_N, _D), jnp.float32),
               jax.ShapeDtypeStruct((_N, _D), jnp.float32)),
)


def _tc3_body(aggp_ref, hp_ref, degp_ref, b_ref, res_ref, wout_ref,
              bout_ref, out_ref):
    dinv = _dinv_block(degp_ref[...])
    agg = aggp_ref[0] + aggp_ref[1]
    z = dinv * (agg + hp_ref[...]) + b_ref[...]
    h = jnp.maximum(z, 0.0) + res_ref[...]
    out_ref[...] = jnp.dot(h, wout_ref[...],
                           preferred_element_type=jnp.float32) + bout_ref[...]


_tc3 = pl.pallas_call(
    _tc3_body,
    grid=(10,),
    in_specs=[pl.BlockSpec((2, 1024, _D), lambda i: (0, i, 0)),
              pl.BlockSpec((1024, _D), lambda i: (i, 0)),
              pl.BlockSpec((2, 8, _D), lambda i: (0, i, 0)),
              pl.BlockSpec((1, _D), lambda i: (0, 0)),
              pl.BlockSpec((1024, _D), lambda i: (i, 0)),
              pl.BlockSpec((_D, _D), lambda i: (0, 0)),
              pl.BlockSpec((1, _D), lambda i: (0, 0))],
    out_specs=pl.BlockSpec((1024, _D), lambda i: (i, 0)),
    out_shape=jax.ShapeDtypeStruct((_N, _D), jnp.float32),
)


def kernel(x, edge_index, W1, b1, W2, b2, W_out, b_out):
    ei = edge_index.reshape(2 * _E)
    dst2 = edge_index[1].reshape(_NW, _EPT)
    dpad = jnp.full((_NW, _CPTD * _CHD - _EPT), _N, jnp.int32)
    dstpad = jnp.concatenate([dst2, dpad], axis=1).reshape(_NW, _CPTD, _CHD)

    degp = _deg_kernel(dstpad)
    h1raw = _tc1a(x, W1)

    degp_r = degp.reshape(_NC, _NPAD // _D, _D)
    hp1 = _tc1b(h1raw, degp_r)
    agg1 = _agg_kernel(hp1, ei)
    h1, hp2 = _tc2(agg1, hp1, degp_r, b1.reshape(1, _D), x, W2)
    agg2 = _agg_kernel(hp2, ei)
    out = _tc3(agg2, hp2, degp_r, b2.reshape(1, _D), h1, W_out,
               b_out.reshape(1, _D))
    return out
